# baseline (speedup 1.0000x reference)
import functools

import jax
import jax.numpy as jnp
from jax import lax
from jax.experimental import pallas as pl
from jax.experimental.pallas import tpu as pltpu
from jax.experimental.pallas import tpu_sc as plsc

B = 4096
L = 50
EMB = 128

_info = plsc.get_sparse_core_info()
NC = _info.num_cores
NS = _info.num_subcores
NW = NC * NS
RPW = B // NW
RPG = 2
NBUF = 8
NOUT = RPW // (RPG * NBUF)

_mesh = plsc.VectorSubcoreMesh(core_axis_name="c", subcore_axis_name="s")


@functools.partial(
    pl.kernel,
    mesh=_mesh,
    out_type=jax.ShapeDtypeStruct((B, L, EMB), jnp.float32),
    scratch_types=(
        [pltpu.VMEM((RPW // RPG, RPG * L), jnp.int32)]
        + [pltpu.VMEM((RPG * L, EMB), jnp.float32) for _ in range(NBUF)]
        + [pltpu.SemaphoreType.DMA for _ in range(2 * NBUF)]
    ),
    compiler_params=pltpu.CompilerParams(use_tc_tiling_on_sc=True, needs_layout_passes=False),
)
def _embed(xf_hbm, w_hbm, out_hbm, idx, *bufs_and_sems):
    rows = bufs_and_sems[:NBUF]
    gsem = bufs_and_sems[NBUF:2 * NBUF]
    ssem = bufs_and_sems[2 * NBUF:]

    wid = lax.axis_index("s") * NC + lax.axis_index("c")
    xr0 = wid * RPW
    pltpu.sync_copy(xf_hbm.at[pl.ds(wid * (RPW // RPG), RPW // RPG), :], idx)

    def gather(i, b):
        pltpu.async_copy(w_hbm.at[idx.at[i]], rows[b], gsem[b])

    def gather_wait(b):
        pltpu.make_async_copy(w_hbm.at[idx.at[0]], rows[b], gsem[b]).wait()

    def store(i, b):
        for r in range(RPG):
            pltpu.async_copy(
                rows[b].at[pl.ds(r * L, L)], out_hbm.at[xr0 + i * RPG + r], ssem[b]
            )

    def store_wait(b):
        for _ in range(RPG):
            pltpu.make_async_copy(rows[b].at[pl.ds(0, L)], out_hbm.at[xr0], ssem[b]).wait()

    for b in range(NBUF):
        gather(b, b)

    def body(it, carry):
        i0 = it * NBUF
        for b in range(NBUF):
            gather_wait(b)
            store(i0 + b, b)
        for b in range(NBUF):
            @pl.when(it < NOUT - 1)
            def _():
                store_wait(b)
                gather(i0 + NBUF + b, b)
        return carry

    lax.fori_loop(0, NOUT, body, 0)

    for b in range(NBUF):
        store_wait(b)


def kernel(x, weight):
    return _embed(x.reshape(B // RPG, RPG * L), weight)
